# baseline (speedup 1.0000x reference)
import functools

import jax
import jax.numpy as jnp
from jax import lax
from jax.experimental import pallas as pl
from jax.experimental.pallas import tpu as pltpu
from jax.experimental.pallas import tpu_sc as plsc

B, S = 16384, 200
D = 64
H, O = 128, 64

NC, NS = 2, 16
NW = NC * NS
BPW = B // NW
NB = 4
CHUNK = S
NCHUNK = BPW
LOOK = NB - 1
UNROLL = 4


def _sc_body(x_hbm, table_hbm, pooled_hbm, idxs, rowss, pooled_v, gsems):
    c_id = lax.axis_index("c")
    s_id = lax.axis_index("s")
    wid = s_id * NC + c_id
    xbase = wid * (BPW * S)

    def _stage_idx(c, idx_v):
        pltpu.sync_copy(x_hbm.at[pl.ds(xbase + c * CHUNK, CHUNK)], idx_v)

    _G_OFFS = [g * 16 for g in range(CHUNK // 16)] + [CHUNK - 16]

    def _fire_gather(idx_v, rows_v, sem):
        for off in _G_OFFS:
            ivec = idx_v[pl.ds(off, 16)]
            pltpu.async_copy(table_hbm.at[ivec], rows_v.at[pl.ds(off, 16)],
                             sem)

    def _drain_gather(idx_v, rows_v, sem):
        for off in _G_OFFS:
            ivec = idx_v[pl.ds(off, 16)]
            pltpu.make_async_copy(table_hbm.at[ivec],
                                  rows_v.at[pl.ds(off, 16)], sem).wait()

    zero16 = jnp.zeros((16,), jnp.float32)

    def _accumulate(c, rows_v):
        def step(t, accs):
            accs = list(accs)
            for u in range(UNROLL):
                i = t * UNROLL + u
                for k in range(D // 16):
                    accs[(u % 2) * 4 + k] = accs[(u % 2) * 4 + k] + \
                        rows_v[i, pl.ds(k * 16, 16)]
            return tuple(accs)

        accs = lax.fori_loop(0, CHUNK // UNROLL, step,
                             tuple(zero16 for _ in range(8)))
        for k in range(D // 16):
            pooled_v[c, pl.ds(k * 16, 16)] = accs[k] + accs[4 + k]

    for k in range(LOOK):
        _stage_idx(k, idxs[k])
        _fire_gather(idxs[k], rowss[k], gsems[k])

    def _rev(rr, carry):
        for k in range(NB):
            c = rr * NB + k
            b = k
            fb = (k + LOOK) % NB

            @pl.when(c + LOOK < NCHUNK)
            def _():
                _stage_idx(c + LOOK, idxs[fb])
                _fire_gather(idxs[fb], rowss[fb], gsems[fb])

            _drain_gather(idxs[b], rowss[b], gsems[b])
            _accumulate(c, rowss[b])
        return carry

    lax.fori_loop(0, NCHUNK // NB, _rev, 0)

    pltpu.sync_copy(pooled_v, pooled_hbm.at[pl.ds(wid * BPW, BPW)])


_sc_pool = pl.kernel(
    _sc_body,
    out_type=jax.ShapeDtypeStruct((B, D), jnp.float32),
    mesh=plsc.VectorSubcoreMesh(core_axis_name="c", subcore_axis_name="s"),
    compiler_params=pltpu.CompilerParams(use_tc_tiling_on_sc=False),
    scratch_types=[
        [pltpu.VMEM((CHUNK,), jnp.int32) for _ in range(NB)],
        [pltpu.VMEM((CHUNK, D), jnp.float32) for _ in range(NB)],
        pltpu.VMEM((BPW, D), jnp.float32),
        [pltpu.SemaphoreType.DMA for _ in range(NB)],
    ],
)


def _mlp_body(p_ref, w1_ref, b1_ref, w2_ref, b2_ref, o_ref):
    p = p_ref[...] * (1.0 / S)
    h = jnp.dot(p, w1_ref[...], preferred_element_type=jnp.float32)
    h = h + b1_ref[...]
    o = jnp.dot(h, w2_ref[...], preferred_element_type=jnp.float32)
    o_ref[...] = o + b2_ref[...]


_MLP_BLK = 1024


@functools.partial(jax.jit, static_argnums=())
def _mlp(pooled, W1, b1, W2, b2):
    return pl.pallas_call(
        _mlp_body,
        grid=(B // _MLP_BLK,),
        in_specs=[
            pl.BlockSpec((_MLP_BLK, D), lambda i: (i, 0)),
            pl.BlockSpec((D, H), lambda i: (0, 0)),
            pl.BlockSpec((1, H), lambda i: (0, 0)),
            pl.BlockSpec((H, O), lambda i: (0, 0)),
            pl.BlockSpec((1, O), lambda i: (0, 0)),
        ],
        out_specs=pl.BlockSpec((_MLP_BLK, O), lambda i: (i, 0)),
        out_shape=jax.ShapeDtypeStruct((B, O), jnp.float32),
    )(pooled, W1, b1, W2, b2)


def kernel(x, table, W1, b1, W2, b2):
    pooled = _sc_pool(x.reshape(B * S), table)
    return _mlp(pooled, W1, b1.reshape(1, H), W2, b2.reshape(1, O))


# speedup vs baseline: 1.0323x; 1.0323x over previous
import functools

import jax
import jax.numpy as jnp
from jax import lax
from jax.experimental import pallas as pl
from jax.experimental.pallas import tpu as pltpu
from jax.experimental.pallas import tpu_sc as plsc

B, S = 16384, 200
D = 64
H, O = 128, 64

NC, NS = 2, 16
NW = NC * NS
BPW = B // NW
NB = 4
CHUNK = S
NCHUNK = BPW
LOOK = NB - 1
UNROLL = 4


def _sc_body(x_hbm, table_hbm, pooled_hbm, idxs, rowss, pooled_v, gsems):
    c_id = lax.axis_index("c")
    s_id = lax.axis_index("s")
    wid = s_id * NC + c_id
    xbase = wid * (BPW * S)

    def _stage_idx(c, idx_v):
        pltpu.sync_copy(x_hbm.at[pl.ds(xbase + c * CHUNK, CHUNK)], idx_v)

    def _fire_gather(idx_v, rows_v, sem):
        pltpu.async_copy(table_hbm.at[idx_v], rows_v, sem)

    def _drain_gather(idx_v, rows_v, sem):
        pltpu.make_async_copy(table_hbm.at[idx_v], rows_v, sem).wait()

    zero16 = jnp.zeros((16,), jnp.float32)

    def _accumulate(c, rows_v):
        def step(t, accs):
            accs = list(accs)
            for u in range(UNROLL):
                i = t * UNROLL + u
                for k in range(D // 16):
                    accs[(u % 2) * 4 + k] = accs[(u % 2) * 4 + k] + \
                        rows_v[i, pl.ds(k * 16, 16)]
            return tuple(accs)

        accs = lax.fori_loop(0, CHUNK // UNROLL, step,
                             tuple(zero16 for _ in range(8)))
        for k in range(D // 16):
            pooled_v[c, pl.ds(k * 16, 16)] = accs[k] + accs[4 + k]

    for k in range(LOOK):
        _stage_idx(k, idxs[k])
        _fire_gather(idxs[k], rowss[k], gsems[k])

    def _rev(rr, carry):
        for k in range(NB):
            c = rr * NB + k
            b = k
            fb = (k + LOOK) % NB

            @pl.when(c + LOOK < NCHUNK)
            def _():
                _stage_idx(c + LOOK, idxs[fb])
                _fire_gather(idxs[fb], rowss[fb], gsems[fb])

            _drain_gather(idxs[b], rowss[b], gsems[b])
            _accumulate(c, rowss[b])
        return carry

    lax.fori_loop(0, NCHUNK // NB, _rev, 0)

    pltpu.sync_copy(pooled_v, pooled_hbm.at[pl.ds(wid * BPW, BPW)])


_sc_pool = pl.kernel(
    _sc_body,
    out_type=jax.ShapeDtypeStruct((B, D), jnp.float32),
    mesh=plsc.VectorSubcoreMesh(core_axis_name="c", subcore_axis_name="s"),
    compiler_params=pltpu.CompilerParams(use_tc_tiling_on_sc=False),
    scratch_types=[
        [pltpu.VMEM((CHUNK,), jnp.int32) for _ in range(NB)],
        [pltpu.VMEM((CHUNK, D), jnp.float32) for _ in range(NB)],
        pltpu.VMEM((BPW, D), jnp.float32),
        [pltpu.SemaphoreType.DMA for _ in range(NB)],
    ],
)


def _mlp_body(p_ref, w1_ref, b1_ref, w2_ref, b2_ref, o_ref):
    p = p_ref[...] * (1.0 / S)
    h = jnp.dot(p, w1_ref[...], preferred_element_type=jnp.float32)
    h = h + b1_ref[...]
    o = jnp.dot(h, w2_ref[...], preferred_element_type=jnp.float32)
    o_ref[...] = o + b2_ref[...]


_MLP_BLK = 1024


@functools.partial(jax.jit, static_argnums=())
def _mlp(pooled, W1, b1, W2, b2):
    return pl.pallas_call(
        _mlp_body,
        grid=(B // _MLP_BLK,),
        in_specs=[
            pl.BlockSpec((_MLP_BLK, D), lambda i: (i, 0)),
            pl.BlockSpec((D, H), lambda i: (0, 0)),
            pl.BlockSpec((1, H), lambda i: (0, 0)),
            pl.BlockSpec((H, O), lambda i: (0, 0)),
            pl.BlockSpec((1, O), lambda i: (0, 0)),
        ],
        out_specs=pl.BlockSpec((_MLP_BLK, O), lambda i: (i, 0)),
        out_shape=jax.ShapeDtypeStruct((B, O), jnp.float32),
    )(pooled, W1, b1, W2, b2)


def kernel(x, table, W1, b1, W2, b2):
    pooled = _sc_pool(x.reshape(B * S), table)
    return _mlp(pooled, W1, b1.reshape(1, H), W2, b2.reshape(1, O))
